# Initial kernel scaffold; baseline (speedup 1.0000x reference)
import jax
import jax.numpy as jnp
from jax.experimental import pallas as pl


def kernel(x_s, x_v, edge_index, edge_s, edge_v, params):
    raise NotImplementedError("write your pallas kernel here")


# speedup vs baseline: 1.6529x; 1.6529x over previous
import functools

import jax
import jax.numpy as jnp
import numpy as np
from jax.experimental import pallas as pl
from jax.experimental.pallas import tpu as pltpu

SD, VD = 32, 4
ES, EV = 32, 1
EPS = 1e-8
LN_EPS = 1e-5


def _expand_wh_y(wh):
    return jnp.kron(jnp.eye(3, dtype=wh.dtype), wh)


def _expand_wh_rowmajor(wh, vi, h):
    p_in = np.zeros((3 * vi, vi, 3), np.float32)
    for i in range(vi):
        for c in range(3):
            p_in[i * 3 + c, i, c] = 1.0
    q_out = np.zeros((3 * h, h, 3), np.float32)
    for k in range(h):
        for c in range(3):
            q_out[c * h + k, k, c] = 1.0
    return jnp.einsum('lic,ik,mkc->lm', p_in, wh, q_out)


def _expand_wv_y(wv):
    return jnp.kron(jnp.eye(3, dtype=wv.dtype), wv)


def _prep_gvp(p, in_layout_rowmajor=False, vi=None):
    wh, wv = p['wh'], p['wv']
    h = wh.shape[1]
    if in_layout_rowmajor:
        whx = _expand_wh_rowmajor(wh, vi, h)
    else:
        whx = _expand_wh_y(wh)
    return {'whx': whx, 'wvx': _expand_wv_y(wv), 'ws_w': p['ws_w'],
            'ws_b': p['ws_b'].reshape(1, -1)}


def _csum3(x2, h):
    return x2[:, :h] + x2[:, h:2 * h] + x2[:, 2 * h:3 * h]


def _gvp_block(s, v, whx, wvx, ws_w, ws_b, h, vo, scalar_act, vector_act):
    vh = jnp.dot(v, whx, preferred_element_type=jnp.float32)
    vn = jnp.sqrt(jnp.maximum(_csum3(vh * vh, h), EPS))
    si = jnp.concatenate([s, vn], axis=-1)
    so = jnp.dot(si, ws_w, preferred_element_type=jnp.float32) + ws_b
    vout = jnp.dot(vh, wvx, preferred_element_type=jnp.float32)
    if vector_act:
        no = jnp.sqrt(jnp.maximum(_csum3(vout * vout, vo), EPS))
        gate = jax.nn.sigmoid(no)
        vout = vout * jnp.concatenate([gate, gate, gate], axis=-1)
    if scalar_act:
        so = jax.nn.relu(so)
    return so, vout


def _layernorm_block(s, v, w, b):
    mu = jnp.mean(s, axis=-1, keepdims=True)
    var = jnp.mean(jnp.square(s - mu), axis=-1, keepdims=True)
    s = (s - mu) * jax.lax.rsqrt(var + LN_EPS) * w + b
    v2 = v * v
    per_i = jnp.maximum(_csum3(v2, VD), EPS)
    vn = jnp.sqrt(jnp.mean(per_i, axis=-1, keepdims=True))
    return s, v / vn


def _input_proj_body(xs_ref, xv_ref, whx, wvx, wsw, wsb, out_ref):
    s, v = _gvp_block(xs_ref[...], xv_ref[...], whx[...], wvx[...],
                      wsw[...], wsb[...], VD, VD, True, True)
    z = jnp.zeros((s.shape[0], 4), jnp.float32)
    out_ref[...] = jnp.concatenate([s, v, z], axis=-1)


def _edge_body(gsrc, gdst, es, ev,
               w1h, w1v, w1s, w1b, w2h, w2v, w2s, w2b, w3h, w3v, w3s, w3b,
               msg_s, msg_v):
    ss = jnp.concatenate([gsrc[:, :SD], es[...], gdst[:, :SD]], axis=-1)
    vs = gsrc[:, SD:SD + 12]
    vd = gdst[:, SD:SD + 12]
    e = ev[...]
    pieces = []
    for c in range(3):
        pieces += [vs[:, c * 4:c * 4 + 4], e[:, c:c + 1], vd[:, c * 4:c * 4 + 4]]
    mv = jnp.concatenate(pieces, axis=-1)
    s1, v1 = _gvp_block(ss, mv, w1h[...], w1v[...], w1s[...], w1b[...],
                        9, VD, True, True)
    s2, v2 = _gvp_block(s1, v1, w2h[...], w2v[...], w2s[...], w2b[...],
                        VD, VD, True, True)
    s3, v3 = _gvp_block(s2, v2, w3h[...], w3v[...], w3s[...], w3b[...],
                        VD, VD, False, False)
    msg_s[...] = s3
    z = jnp.zeros((s3.shape[0], 4), jnp.float32)
    msg_v[...] = jnp.concatenate([v3, z], axis=-1)


def _node_body(nf, aggs, aggv, cnt,
               ln0w, ln0b, f1h, f1v, f1s, f1b, f2h, f2v, f2s, f2b,
               ln1w, ln1b, lnow, lnob, out_ref, *, final):
    s = nf[:, :SD]
    v = nf[:, SD:SD + 12]
    c = jnp.maximum(cnt[:, 0:1], 1.0)
    s = s + aggs[...] / c
    v = v + aggv[:, :12] / c
    s, v = _layernorm_block(s, v, ln0w[...], ln0b[...])
    fs, fv = _gvp_block(s, v, f1h[...], f1v[...], f1s[...], f1b[...],
                        8, 8, True, True)
    fs, fv = _gvp_block(fs, fv, f2h[...], f2v[...], f2s[...], f2b[...],
                        8, VD, False, False)
    s, v = _layernorm_block(s + fs, v + fv, ln1w[...], ln1b[...])
    if final:
        s, v = _layernorm_block(s, v, lnow[...], lnob[...])
    z = jnp.zeros((s.shape[0], 4), jnp.float32)
    out_ref[...] = jnp.concatenate([s, v, z], axis=-1)


def _full_spec(shape):
    nd = len(shape)
    return pl.BlockSpec(shape, lambda i: (0,) * nd)


def _row_spec(bn, w):
    return pl.BlockSpec((bn, w), lambda i: (i, 0))


def _tc_call(body, grid, in_specs, out_specs, out_shape):
    return pl.pallas_call(
        body, grid=(grid,), in_specs=in_specs, out_specs=out_specs,
        out_shape=out_shape)


def kernel(x_s, x_v, edge_index, edge_s, edge_v, params):
    n = x_s.shape[0]
    e = edge_index.shape[1]
    src, dst = edge_index[0], edge_index[1]

    bn = 4000
    be = 4000
    assert n % bn == 0 and e % be == 0

    pin = params['input_proj']
    pin_whx = _expand_wh_rowmajor(pin['wh'], 3, VD)
    pin_wvx = _expand_wv_y(pin['wv'])
    xv9 = x_v.reshape(n, 9)
    nf = _tc_call(
        _input_proj_body, n // bn,
        [_row_spec(bn, 6), _row_spec(bn, 9),
         _full_spec(pin_whx.shape), _full_spec(pin_wvx.shape),
         _full_spec(pin['ws_w'].shape), _full_spec((1, SD))],
        _row_spec(bn, 48),
        jax.ShapeDtypeStruct((n, 48), jnp.float32),
    )(x_s, xv9, pin_whx, pin_wvx, pin['ws_w'], pin['ws_b'].reshape(1, SD))

    ones = jnp.ones((e,), jnp.float32)
    cnt = jax.ops.segment_sum(ones, dst, num_segments=n).reshape(n, 1)
    cnt8 = jnp.pad(cnt, ((0, 0), (0, 7)))

    for li, lp in enumerate(params['layers']):
        g1 = _prep_gvp(lp['msg'][0])
        g2 = _prep_gvp(lp['msg'][1])
        g3 = _prep_gvp(lp['msg'][2])
        f1 = _prep_gvp(lp['ff'][0])
        f2 = _prep_gvp(lp['ff'][1])

        gsrc = nf[src]
        gdst = nf[dst]

        ev3 = edge_v.reshape(e, 3)
        wspecs = []
        wvals = []
        for g in (g1, g2, g3):
            for k in ('whx', 'wvx', 'ws_w', 'ws_b'):
                wspecs.append(_full_spec(g[k].shape))
                wvals.append(g[k])
        msg_s, msg_v = _tc_call(
            _edge_body, e // be,
            [_row_spec(be, 48), _row_spec(be, 48), _row_spec(be, ES),
             _row_spec(be, 3)] + wspecs,
            [_row_spec(be, SD), _row_spec(be, 16)],
            [jax.ShapeDtypeStruct((e, SD), jnp.float32),
             jax.ShapeDtypeStruct((e, 16), jnp.float32)],
        )(gsrc, gdst, edge_s, ev3, *wvals)

        aggs = jax.ops.segment_sum(msg_s, dst, num_segments=n)
        aggv = jax.ops.segment_sum(msg_v, dst, num_segments=n)

        final = li == 2
        node_w = [lp['ln0_w'].reshape(1, SD), lp['ln0_b'].reshape(1, SD),
                  f1['whx'], f1['wvx'], f1['ws_w'], f1['ws_b'],
                  f2['whx'], f2['wvx'], f2['ws_w'], f2['ws_b'],
                  lp['ln1_w'].reshape(1, SD), lp['ln1_b'].reshape(1, SD),
                  params['ln_out_w'].reshape(1, SD),
                  params['ln_out_b'].reshape(1, SD)]
        nf = _tc_call(
            functools.partial(_node_body, final=final), n // bn,
            [_row_spec(bn, 48), _row_spec(bn, SD), _row_spec(bn, 16),
             _row_spec(bn, 8)] + [_full_spec(w.shape) for w in node_w],
            _row_spec(bn, 48),
            jax.ShapeDtypeStruct((n, 48), jnp.float32),
        )(nf, aggs, aggv, cnt8, *node_w)

    s_out = nf[:, :SD]
    v_out = nf[:, SD:SD + 12].reshape(n, 3, VD).swapaxes(1, 2)
    return s_out, v_out


# speedup vs baseline: 1.7217x; 1.0416x over previous
import functools

import jax
import jax.numpy as jnp
import numpy as np
from jax import lax
from jax.experimental import pallas as pl
from jax.experimental.pallas import tpu as pltpu
from jax.experimental.pallas import tpu_sc as plsc

NC, NS = 2, 16
NW = NC * NS
CH = 80

SD, VD = 32, 4
ES, EV = 32, 1
EPS = 1e-8
LN_EPS = 1e-5


def _expand_wh_y(wh):
    return jnp.kron(jnp.eye(3, dtype=wh.dtype), wh)


def _expand_wh_rowmajor(wh, vi, h):
    p_in = np.zeros((3 * vi, vi, 3), np.float32)
    for i in range(vi):
        for c in range(3):
            p_in[i * 3 + c, i, c] = 1.0
    q_out = np.zeros((3 * h, h, 3), np.float32)
    for k in range(h):
        for c in range(3):
            q_out[c * h + k, k, c] = 1.0
    return jnp.einsum('lic,ik,mkc->lm', p_in, wh, q_out)


def _expand_wv_y(wv):
    return jnp.kron(jnp.eye(3, dtype=wv.dtype), wv)


def _prep_gvp(p, in_layout_rowmajor=False, vi=None):
    wh, wv = p['wh'], p['wv']
    h = wh.shape[1]
    if in_layout_rowmajor:
        whx = _expand_wh_rowmajor(wh, vi, h)
    else:
        whx = _expand_wh_y(wh)
    return {'whx': whx, 'wvx': _expand_wv_y(wv), 'ws_w': p['ws_w'],
            'ws_b': p['ws_b'].reshape(1, -1)}


def _csum3(x2, h):
    return x2[:, :h] + x2[:, h:2 * h] + x2[:, 2 * h:3 * h]


def _gvp_block(s, v, whx, wvx, ws_w, ws_b, h, vo, scalar_act, vector_act):
    vh = jnp.dot(v, whx, preferred_element_type=jnp.float32)
    vn = jnp.sqrt(jnp.maximum(_csum3(vh * vh, h), EPS))
    si = jnp.concatenate([s, vn], axis=-1)
    so = jnp.dot(si, ws_w, preferred_element_type=jnp.float32) + ws_b
    vout = jnp.dot(vh, wvx, preferred_element_type=jnp.float32)
    if vector_act:
        no = jnp.sqrt(jnp.maximum(_csum3(vout * vout, vo), EPS))
        gate = jax.nn.sigmoid(no)
        vout = vout * jnp.concatenate([gate, gate, gate], axis=-1)
    if scalar_act:
        so = jax.nn.relu(so)
    return so, vout


def _layernorm_block(s, v, w, b):
    mu = jnp.mean(s, axis=-1, keepdims=True)
    var = jnp.mean(jnp.square(s - mu), axis=-1, keepdims=True)
    s = (s - mu) * jax.lax.rsqrt(var + LN_EPS) * w + b
    v2 = v * v
    per_i = jnp.maximum(_csum3(v2, VD), EPS)
    vn = jnp.sqrt(jnp.mean(per_i, axis=-1, keepdims=True))
    return s, v / vn


def _input_proj_body(xs_ref, xv_ref, whx, wvx, wsw, wsb, out_ref):
    s, v = _gvp_block(xs_ref[...], xv_ref[...], whx[...], wvx[...],
                      wsw[...], wsb[...], VD, VD, True, True)
    z = jnp.zeros((s.shape[0], 128 - SD - 12), jnp.float32)
    out_ref[...] = jnp.concatenate([s, v, z], axis=-1)


def _edge_body(gsrc, gdst, es, ev,
               w1h, w1v, w1s, w1b, w2h, w2v, w2s, w2b, w3h, w3v, w3s, w3b,
               msg_s, msg_v):
    ss = jnp.concatenate([gsrc[:, :SD], es[...], gdst[:, :SD]], axis=-1)
    vs = gsrc[:, SD:SD + 12]
    vd = gdst[:, SD:SD + 12]
    e = ev[...]
    pieces = []
    for c in range(3):
        pieces += [vs[:, c * 4:c * 4 + 4], e[:, c:c + 1], vd[:, c * 4:c * 4 + 4]]
    mv = jnp.concatenate(pieces, axis=-1)
    s1, v1 = _gvp_block(ss, mv, w1h[...], w1v[...], w1s[...], w1b[...],
                        9, VD, True, True)
    s2, v2 = _gvp_block(s1, v1, w2h[...], w2v[...], w2s[...], w2b[...],
                        VD, VD, True, True)
    s3, v3 = _gvp_block(s2, v2, w3h[...], w3v[...], w3s[...], w3b[...],
                        VD, VD, False, False)
    msg_s[...] = s3
    z = jnp.zeros((s3.shape[0], 4), jnp.float32)
    msg_v[...] = jnp.concatenate([v3, z], axis=-1)


def _node_body(nf, aggs, aggv, cnt,
               ln0w, ln0b, f1h, f1v, f1s, f1b, f2h, f2v, f2s, f2b,
               ln1w, ln1b, lnow, lnob, out_ref, *, final):
    s = nf[:, :SD]
    v = nf[:, SD:SD + 12]
    c = jnp.maximum(cnt[:, 0:1], 1.0)
    s = s + aggs[...] / c
    v = v + aggv[:, :12] / c
    s, v = _layernorm_block(s, v, ln0w[...], ln0b[...])
    fs, fv = _gvp_block(s, v, f1h[...], f1v[...], f1s[...], f1b[...],
                        8, 8, True, True)
    fs, fv = _gvp_block(fs, fv, f2h[...], f2v[...], f2s[...], f2b[...],
                        8, VD, False, False)
    s, v = _layernorm_block(s + fs, v + fv, ln1w[...], ln1b[...])
    if final:
        s, v = _layernorm_block(s, v, lnow[...], lnob[...])
    z = jnp.zeros((s.shape[0], 128 - SD - 12), jnp.float32)
    out_ref[...] = jnp.concatenate([s, v, z], axis=-1)


def _full_spec(shape):
    nd = len(shape)
    return pl.BlockSpec(shape, lambda i: (0,) * nd)


def _row_spec(bn, w):
    return pl.BlockSpec((bn, w), lambda i: (i, 0))


def _tc_call(body, grid, in_specs, out_specs, out_shape):
    return pl.pallas_call(
        body, grid=(grid,), in_specs=in_specs, out_specs=out_specs,
        out_shape=out_shape)


_SC_MESH = dict(core_axis_name="c", subcore_axis_name="s",
                num_cores=NC, num_subcores=NS)


def _make_gather(n, e, grp):
    w = 128
    per_tile = e // NW
    chunks_pt = per_tile // CH
    iters = chunks_pt // grp
    assert per_tile % CH == 0 and chunks_pt % grp == 0
    assert chunks_pt % 8 == 0 and grp % 8 == 0
    rows_g = grp * CH

    def body(table, idx2d, out, idxb, rowsb, sem_g, sem_w):
        wid = lax.axis_index("s") * NC + lax.axis_index("c")

        def step(g, _):
            coff = wid * chunks_pt + g * grp
            pltpu.sync_copy(idx2d.at[pl.ds(coff, grp)], idxb)
            descs = [
                pltpu.async_copy(table.at[idxb.at[j]],
                                 rowsb.at[pl.ds(j * CH, CH)], sem_g)
                for j in range(grp)
            ]
            for d in descs:
                d.wait()
            pltpu.async_copy(
                rowsb, out.at[pl.ds(wid * per_tile + g * rows_g, rows_g)],
                sem_w).wait()
            return 0

        lax.fori_loop(0, iters, step, 0)

    return pl.kernel(
        body,
        out_type=jax.ShapeDtypeStruct((e, w), jnp.float32),
        mesh=plsc.VectorSubcoreMesh(**_SC_MESH),
        scratch_types=[
            pltpu.VMEM((grp, CH), jnp.int32),
            pltpu.VMEM((rows_g, w), jnp.float32),
            pltpu.SemaphoreType.DMA,
            pltpu.SemaphoreType.DMA,
        ])


def kernel(x_s, x_v, edge_index, edge_s, edge_v, params):
    n = x_s.shape[0]
    e = edge_index.shape[1]
    src, dst = edge_index[0], edge_index[1]

    bn = 4000
    be = 4000
    assert n % bn == 0 and e % be == 0

    pin = params['input_proj']
    pin_whx = _expand_wh_rowmajor(pin['wh'], 3, VD)
    pin_wvx = _expand_wv_y(pin['wv'])
    xv9 = x_v.reshape(n, 9)
    nf = _tc_call(
        _input_proj_body, n // bn,
        [_row_spec(bn, 6), _row_spec(bn, 9),
         _full_spec(pin_whx.shape), _full_spec(pin_wvx.shape),
         _full_spec(pin['ws_w'].shape), _full_spec((1, SD))],
        _row_spec(bn, 128),
        jax.ShapeDtypeStruct((n, 128), jnp.float32),
    )(x_s, xv9, pin_whx, pin_wvx, pin['ws_w'], pin['ws_b'].reshape(1, SD))

    ep = 1638400
    assert e <= ep
    src_p = jnp.concatenate([src, jnp.zeros((ep - e,), jnp.int32)])
    dst_p = jnp.concatenate([dst, jnp.full((ep - e,), n, jnp.int32)])
    src2d = src_p.reshape(ep // CH, CH)
    dst2d = dst_p.reshape(ep // CH, CH)
    grp = 8
    gather128 = _make_gather(n, ep, grp)

    cnt8 = jnp.pad(jax.ops.segment_sum(jnp.ones((e,), jnp.float32), dst,
                                       num_segments=n).reshape(n, 1),
                   ((0, 0), (0, 7)))

    for li, lp in enumerate(params['layers']):
        g1 = _prep_gvp(lp['msg'][0])
        g2 = _prep_gvp(lp['msg'][1])
        g3 = _prep_gvp(lp['msg'][2])
        f1 = _prep_gvp(lp['ff'][0])
        f2 = _prep_gvp(lp['ff'][1])

        gsrc = gather128(nf, src2d)
        gdst = gather128(nf, dst2d)

        ev3 = edge_v.reshape(e, 3)
        wspecs = []
        wvals = []
        for g in (g1, g2, g3):
            for k in ('whx', 'wvx', 'ws_w', 'ws_b'):
                wspecs.append(_full_spec(g[k].shape))
                wvals.append(g[k])
        msg_s, msg_v = _tc_call(
            _edge_body, e // be,
            [_row_spec(be, 128), _row_spec(be, 128), _row_spec(be, ES),
             _row_spec(be, 3)] + wspecs,
            [_row_spec(be, SD), _row_spec(be, 16)],
            [jax.ShapeDtypeStruct((e, SD), jnp.float32),
             jax.ShapeDtypeStruct((e, 16), jnp.float32)],
        )(gsrc, gdst, edge_s, ev3, *wvals)

        aggs = jax.ops.segment_sum(msg_s, dst, num_segments=n)
        aggv = jax.ops.segment_sum(msg_v, dst, num_segments=n)

        final = li == 2
        node_w = [lp['ln0_w'].reshape(1, SD), lp['ln0_b'].reshape(1, SD),
                  f1['whx'], f1['wvx'], f1['ws_w'], f1['ws_b'],
                  f2['whx'], f2['wvx'], f2['ws_w'], f2['ws_b'],
                  lp['ln1_w'].reshape(1, SD), lp['ln1_b'].reshape(1, SD),
                  params['ln_out_w'].reshape(1, SD),
                  params['ln_out_b'].reshape(1, SD)]
        nf = _tc_call(
            functools.partial(_node_body, final=final), n // bn,
            [_row_spec(bn, 128), _row_spec(bn, SD), _row_spec(bn, 16),
             _row_spec(bn, 8)] + [_full_spec(w.shape) for w in node_w],
            _row_spec(bn, 128),
            jax.ShapeDtypeStruct((n, 128), jnp.float32),
        )(nf, aggs, aggv, cnt8, *node_w)

    s_out = nf[:, :SD]
    v_out = nf[:, SD:SD + 12].reshape(n, 3, VD).swapaxes(1, 2)
    return s_out, v_out


# speedup vs baseline: 2.5928x; 1.5059x over previous
import functools

import jax
import jax.numpy as jnp
import numpy as np
from jax import lax
from jax.experimental import pallas as pl
from jax.experimental.pallas import tpu as pltpu
from jax.experimental.pallas import tpu_sc as plsc

NC, NS = 2, 16
NW = NC * NS
CH = 80

SD, VD = 32, 4
ES, EV = 32, 1
EPS = 1e-8
LN_EPS = 1e-5


def _expand_wh_y(wh):
    return jnp.kron(jnp.eye(3, dtype=wh.dtype), wh)


def _expand_wh_rowmajor(wh, vi, h):
    p_in = np.zeros((3 * vi, vi, 3), np.float32)
    for i in range(vi):
        for c in range(3):
            p_in[i * 3 + c, i, c] = 1.0
    q_out = np.zeros((3 * h, h, 3), np.float32)
    for k in range(h):
        for c in range(3):
            q_out[c * h + k, k, c] = 1.0
    return jnp.einsum('lic,ik,mkc->lm', p_in, wh, q_out)


def _expand_wv_y(wv):
    return jnp.kron(jnp.eye(3, dtype=wv.dtype), wv)


def _prep_gvp(p, in_layout_rowmajor=False, vi=None):
    wh, wv = p['wh'], p['wv']
    h = wh.shape[1]
    if in_layout_rowmajor:
        whx = _expand_wh_rowmajor(wh, vi, h)
    else:
        whx = _expand_wh_y(wh)
    return {'whx': whx, 'wvx': _expand_wv_y(wv), 'ws_w': p['ws_w'],
            'ws_b': p['ws_b'].reshape(1, -1)}


def _csum3(x2, h):
    return x2[:, :h] + x2[:, h:2 * h] + x2[:, 2 * h:3 * h]


def _gvp_block(s, v, whx, wvx, ws_w, ws_b, h, vo, scalar_act, vector_act):
    vh = jnp.dot(v, whx, preferred_element_type=jnp.float32)
    vn = jnp.sqrt(jnp.maximum(_csum3(vh * vh, h), EPS))
    si = jnp.concatenate([s, vn], axis=-1)
    so = jnp.dot(si, ws_w, preferred_element_type=jnp.float32) + ws_b
    vout = jnp.dot(vh, wvx, preferred_element_type=jnp.float32)
    if vector_act:
        no = jnp.sqrt(jnp.maximum(_csum3(vout * vout, vo), EPS))
        gate = jax.nn.sigmoid(no)
        vout = vout * jnp.concatenate([gate, gate, gate], axis=-1)
    if scalar_act:
        so = jax.nn.relu(so)
    return so, vout


def _layernorm_block(s, v, w, b):
    mu = jnp.mean(s, axis=-1, keepdims=True)
    var = jnp.mean(jnp.square(s - mu), axis=-1, keepdims=True)
    s = (s - mu) * jax.lax.rsqrt(var + LN_EPS) * w + b
    v2 = v * v
    per_i = jnp.maximum(_csum3(v2, VD), EPS)
    vn = jnp.sqrt(jnp.mean(per_i, axis=-1, keepdims=True))
    return s, v / vn


def _input_proj_body(xs_ref, xv_ref, whx, wvx, wsw, wsb, out_ref):
    s, v = _gvp_block(xs_ref[...], xv_ref[...], whx[...], wvx[...],
                      wsw[...], wsb[...], VD, VD, True, True)
    z = jnp.zeros((s.shape[0], 128 - SD - 12), jnp.float32)
    out_ref[...] = jnp.concatenate([s, v, z], axis=-1)


def _edge_body(gsrc, gdst, es, ev,
               w1h, w1v, w1s, w1b, w2h, w2v, w2s, w2b, w3h, w3v, w3s, w3b,
               msg_s):
    ss = jnp.concatenate([gsrc[:, :SD], es[...], gdst[:, :SD]], axis=-1)
    vs = gsrc[:, SD:SD + 12]
    vd = gdst[:, SD:SD + 12]
    e = ev[...]
    pieces = []
    for c in range(3):
        pieces += [vs[:, c * 4:c * 4 + 4], e[:, c:c + 1], vd[:, c * 4:c * 4 + 4]]
    mv = jnp.concatenate(pieces, axis=-1)
    s1, v1 = _gvp_block(ss, mv, w1h[...], w1v[...], w1s[...], w1b[...],
                        9, VD, True, True)
    s2, v2 = _gvp_block(s1, v1, w2h[...], w2v[...], w2s[...], w2b[...],
                        VD, VD, True, True)
    s3, v3 = _gvp_block(s2, v2, w3h[...], w3v[...], w3s[...], w3b[...],
                        VD, VD, False, False)
    z = jnp.zeros((s3.shape[0], 4), jnp.float32)
    msg_s[...] = jnp.concatenate([s3, v3, z], axis=-1)


def _node_body(nf, agg, cnt,
               ln0w, ln0b, f1h, f1v, f1s, f1b, f2h, f2v, f2s, f2b,
               ln1w, ln1b, lnow, lnob, out_ref, *, final):
    s = nf[:, :SD]
    v = nf[:, SD:SD + 12]
    c = jnp.maximum(cnt[:, 0:1], 1.0)
    s = s + agg[:, :SD] / c
    v = v + agg[:, SD:SD + 12] / c
    s, v = _layernorm_block(s, v, ln0w[...], ln0b[...])
    fs, fv = _gvp_block(s, v, f1h[...], f1v[...], f1s[...], f1b[...],
                        8, 8, True, True)
    fs, fv = _gvp_block(fs, fv, f2h[...], f2v[...], f2s[...], f2b[...],
                        8, VD, False, False)
    s, v = _layernorm_block(s + fs, v + fv, ln1w[...], ln1b[...])
    if final:
        s, v = _layernorm_block(s, v, lnow[...], lnob[...])
    z = jnp.zeros((s.shape[0], 128 - SD - 12), jnp.float32)
    out_ref[...] = jnp.concatenate([s, v, z], axis=-1)


def _full_spec(shape):
    nd = len(shape)
    return pl.BlockSpec(shape, lambda i: (0,) * nd)


def _row_spec(bn, w):
    return pl.BlockSpec((bn, w), lambda i: (i, 0))


def _tc_call(body, grid, in_specs, out_specs, out_shape):
    return pl.pallas_call(
        body, grid=(grid,), in_specs=in_specs, out_specs=out_specs,
        out_shape=out_shape)


_SC_MESH = dict(core_axis_name="c", subcore_axis_name="s",
                num_cores=NC, num_subcores=NS)


def _make_gather(n, e, grp):
    w = 128
    per_tile = e // NW
    chunks_pt = per_tile // CH
    iters = chunks_pt // grp
    assert per_tile % CH == 0 and chunks_pt % grp == 0
    assert (grp * CH) % 8 == 0
    rows_g = grp * CH

    def body(table, idx, out, idxb, rowsb, sem_g, sem_w):
        wid = lax.axis_index("s") * NC + lax.axis_index("c")

        def step(g, _):
            eoff = wid * per_tile + g * rows_g
            pltpu.sync_copy(idx.at[pl.ds(eoff, rows_g)], idxb)
            descs = [
                pltpu.async_copy(table.at[idxb.at[pl.ds(j * CH, CH)]],
                                 rowsb.at[pl.ds(j * CH, CH)], sem_g)
                for j in range(grp)
            ]
            for d in descs:
                d.wait()
            pltpu.async_copy(rowsb, out.at[pl.ds(eoff, rows_g)],
                             sem_w).wait()
            return 0

        lax.fori_loop(0, iters, step, 0)

    return pl.kernel(
        body,
        out_type=jax.ShapeDtypeStruct((e, w), jnp.float32),
        mesh=plsc.VectorSubcoreMesh(**_SC_MESH),
        scratch_types=[
            pltpu.VMEM((rows_g,), jnp.int32),
            pltpu.VMEM((rows_g, w), jnp.float32),
            pltpu.SemaphoreType.DMA,
            pltpu.SemaphoreType.DMA,
        ])


def kernel(x_s, x_v, edge_index, edge_s, edge_v, params):
    n = x_s.shape[0]
    e = edge_index.shape[1]
    src, dst = edge_index[0], edge_index[1]

    bn = 4000
    be = 4000
    assert n % bn == 0 and e % be == 0

    pin = params['input_proj']
    pin_whx = _expand_wh_rowmajor(pin['wh'], 3, VD)
    pin_wvx = _expand_wv_y(pin['wv'])
    xv9 = x_v.reshape(n, 9)
    nf = _tc_call(
        _input_proj_body, n // bn,
        [_row_spec(bn, 6), _row_spec(bn, 9),
         _full_spec(pin_whx.shape), _full_spec(pin_wvx.shape),
         _full_spec(pin['ws_w'].shape), _full_spec((1, SD))],
        _row_spec(bn, 128),
        jax.ShapeDtypeStruct((n, 128), jnp.float32),
    )(x_s, xv9, pin_whx, pin_wvx, pin['ws_w'], pin['ws_b'].reshape(1, SD))

    gather128 = _make_gather(n, e, 5)

    cnt8 = jnp.pad(jax.ops.segment_sum(jnp.ones((e,), jnp.float32), dst,
                                       num_segments=n).reshape(n, 1),
                   ((0, 0), (0, 7)))

    for li, lp in enumerate(params['layers']):
        g1 = _prep_gvp(lp['msg'][0])
        g2 = _prep_gvp(lp['msg'][1])
        g3 = _prep_gvp(lp['msg'][2])
        f1 = _prep_gvp(lp['ff'][0])
        f2 = _prep_gvp(lp['ff'][1])

        gsrc = gather128(nf, src)
        gdst = gather128(nf, dst)

        ev3 = edge_v.reshape(e, 3)
        wspecs = []
        wvals = []
        for g in (g1, g2, g3):
            for k in ('whx', 'wvx', 'ws_w', 'ws_b'):
                wspecs.append(_full_spec(g[k].shape))
                wvals.append(g[k])
        msg = _tc_call(
            _edge_body, e // be,
            [_row_spec(be, 128), _row_spec(be, 128), _row_spec(be, ES),
             _row_spec(be, 3)] + wspecs,
            _row_spec(be, 48),
            jax.ShapeDtypeStruct((e, 48), jnp.float32),
        )(gsrc, gdst, edge_s, ev3, *wvals)

        agg = jax.ops.segment_sum(msg, dst, num_segments=n)

        final = li == 2
        node_w = [lp['ln0_w'].reshape(1, SD), lp['ln0_b'].reshape(1, SD),
                  f1['whx'], f1['wvx'], f1['ws_w'], f1['ws_b'],
                  f2['whx'], f2['wvx'], f2['ws_w'], f2['ws_b'],
                  lp['ln1_w'].reshape(1, SD), lp['ln1_b'].reshape(1, SD),
                  params['ln_out_w'].reshape(1, SD),
                  params['ln_out_b'].reshape(1, SD)]
        nf = _tc_call(
            functools.partial(_node_body, final=final), n // bn,
            [_row_spec(bn, 128), _row_spec(bn, 48),
             _row_spec(bn, 8)] + [_full_spec(w.shape) for w in node_w],
            _row_spec(bn, 128),
            jax.ShapeDtypeStruct((n, 128), jnp.float32),
        )(nf, agg, cnt8, *node_w)

    s_out = nf[:, :SD]
    v_out = nf[:, SD:SD + 12].reshape(n, 3, VD).swapaxes(1, 2)
    return s_out, v_out
